# Initial kernel scaffold; baseline (speedup 1.0000x reference)
import jax
import jax.numpy as jnp
from jax.experimental import pallas as pl


def kernel(X, W1, b1, W2, b2):
    raise NotImplementedError("write your pallas kernel here")


# speedup vs baseline: 5.2927x; 5.2927x over previous
import jax
import jax.numpy as jnp
from jax.experimental import pallas as pl
from jax.experimental.pallas import tpu as pltpu

_N = 2048
_F = 256
_D = 512
_K = 32
_ALPHA = 3.0
_RB = 256


def _nv_kernel(x_ref, w1_ref, b1_ref, w2_ref, b2_ref, nv1_ref, nv2_ref):
    x = x_ref[0]
    dn = (((1,), (1,)), ((), ()))
    h1 = jax.lax.dot_general(x, w1_ref[...], dn,
                             preferred_element_type=jnp.float32)
    nv1_ref[0] = jnp.tanh(_ALPHA * (h1 + b1_ref[...]))
    h2 = jax.lax.dot_general(x, w2_ref[...], dn,
                             preferred_element_type=jnp.float32)
    nv2_ref[0] = jnp.tanh(_ALPHA * (h2 + b2_ref[...]))


def _adj_kernel(nv1_ref, nv2_ref, out_ref, vbuf):
    i = pl.program_id(1)
    r0 = i * _RB
    nv1 = nv1_ref[0]
    nv2 = nv2_ref[0]
    nv1s = nv1_ref[0, pl.ds(r0, _RB), :]
    nv2s = nv2_ref[0, pl.ds(r0, _RB), :]
    dn = (((1,), (1,)), ((), ()))
    a = jax.lax.dot_general(nv1s, nv2, dn, preferred_element_type=jnp.float32)
    a -= jax.lax.dot_general(nv2s, nv1, dn, preferred_element_type=jnp.float32)
    adj = jnp.maximum(jnp.tanh(_ALPHA * a), 0.0)

    col = jax.lax.broadcasted_iota(jnp.int32, (_RB, _N), 1)
    vbuf[...] = adj
    out_ref[0] = jnp.zeros((_RB, _N), jnp.float32)

    def body(_, carry):
        v = vbuf[...]
        m = jnp.max(v, axis=1, keepdims=True)
        cand = jnp.where(v == m, col, _N)
        amin = jnp.min(cand, axis=1, keepdims=True)
        sel = col == amin
        out_ref[0] = jnp.where(sel, v, out_ref[0])
        vbuf[...] = jnp.where(sel, -1.0, v)
        return carry

    jax.lax.fori_loop(0, _K, body, 0)


def kernel(X, W1, b1, W2, b2):
    B = X.shape[0]
    b1r = b1.reshape(1, _D)
    b2r = b2.reshape(1, _D)

    nv1, nv2 = pl.pallas_call(
        _nv_kernel,
        grid=(B,),
        in_specs=[
            pl.BlockSpec((1, _N, _F), lambda b: (b, 0, 0)),
            pl.BlockSpec((_D, _F), lambda b: (0, 0)),
            pl.BlockSpec((1, _D), lambda b: (0, 0)),
            pl.BlockSpec((_D, _F), lambda b: (0, 0)),
            pl.BlockSpec((1, _D), lambda b: (0, 0)),
        ],
        out_specs=[
            pl.BlockSpec((1, _N, _D), lambda b: (b, 0, 0)),
            pl.BlockSpec((1, _N, _D), lambda b: (b, 0, 0)),
        ],
        out_shape=[
            jax.ShapeDtypeStruct((B, _N, _D), jnp.float32),
            jax.ShapeDtypeStruct((B, _N, _D), jnp.float32),
        ],
    )(X, W1, b1r, W2, b2r)

    adj = pl.pallas_call(
        _adj_kernel,
        grid=(B, _N // _RB),
        in_specs=[
            pl.BlockSpec((1, _N, _D), lambda b, i: (b, 0, 0)),
            pl.BlockSpec((1, _N, _D), lambda b, i: (b, 0, 0)),
        ],
        out_specs=pl.BlockSpec((1, _RB, _N), lambda b, i: (b, i, 0)),
        out_shape=jax.ShapeDtypeStruct((B, _N, _N), jnp.float32),
        scratch_shapes=[pltpu.VMEM((_RB, _N), jnp.float32)],
    )(nv1, nv2)

    return adj


# speedup vs baseline: 19.7395x; 3.7296x over previous
import jax
import jax.numpy as jnp
from jax.experimental import pallas as pl
from jax.experimental.pallas import tpu as pltpu

_N = 2048
_F = 256
_D = 512
_K = 32
_ALPHA = 3.0
_RB = 256


def _nv_kernel(x_ref, w1_ref, b1_ref, w2_ref, b2_ref, nv1_ref, nv2_ref):
    x = x_ref[0]
    dn = (((1,), (1,)), ((), ()))
    h1 = jax.lax.dot_general(x, w1_ref[...], dn,
                             preferred_element_type=jnp.float32)
    nv1_ref[0] = jnp.tanh(_ALPHA * (h1 + b1_ref[...]))
    h2 = jax.lax.dot_general(x, w2_ref[...], dn,
                             preferred_element_type=jnp.float32)
    nv2_ref[0] = jnp.tanh(_ALPHA * (h2 + b2_ref[...]))


def _prefix_count(x):
    sh = 1
    while sh < _N:
        shifted = jnp.concatenate(
            [jnp.zeros((_RB, sh), x.dtype), x[:, :_N - sh]], axis=1)
        x = x + shifted
        sh *= 2
    return x


def _adj_kernel(nv1_ref, nv2_ref, out_ref, vbuf, rem_ref):
    i = pl.program_id(1)
    r0 = i * _RB
    nv1 = nv1_ref[0]
    nv2 = nv2_ref[0]
    nv1s = nv1_ref[0, pl.ds(r0, _RB), :]
    nv2s = nv2_ref[0, pl.ds(r0, _RB), :]
    dn = (((1,), (1,)), ((), ()))
    a = jax.lax.dot_general(nv1s, nv2, dn, preferred_element_type=jnp.float32)
    a -= jax.lax.dot_general(nv2s, nv1, dn, preferred_element_type=jnp.float32)
    adj = jnp.maximum(jnp.tanh(_ALPHA * a), 0.0)

    vbuf[...] = adj
    rem_ref[...] = jnp.full((_RB, 1), _K, jnp.int32)
    out_ref[0] = jnp.zeros((_RB, _N), jnp.float32)

    def cond(tot):
        return tot > 0

    def body(tot):
        v = vbuf[...]
        remv = rem_ref[...]
        m = jnp.max(v, axis=1, keepdims=True)
        eq = v == m
        pc = _prefix_count(eq.astype(jnp.int32))
        cnt = pc[:, _N - 1:_N]
        take = eq & (pc <= remv)
        out_ref[0] = jnp.where(take, v, out_ref[0])
        vbuf[...] = jnp.where(take, -1.0, v)
        rem_new = remv - jnp.minimum(cnt, remv)
        rem_ref[...] = rem_new
        return jnp.sum(rem_new)

    jax.lax.while_loop(cond, body, jnp.int32(_RB * _K))


def kernel(X, W1, b1, W2, b2):
    B = X.shape[0]
    b1r = b1.reshape(1, _D)
    b2r = b2.reshape(1, _D)

    nv1, nv2 = pl.pallas_call(
        _nv_kernel,
        grid=(B,),
        in_specs=[
            pl.BlockSpec((1, _N, _F), lambda b: (b, 0, 0)),
            pl.BlockSpec((_D, _F), lambda b: (0, 0)),
            pl.BlockSpec((1, _D), lambda b: (0, 0)),
            pl.BlockSpec((_D, _F), lambda b: (0, 0)),
            pl.BlockSpec((1, _D), lambda b: (0, 0)),
        ],
        out_specs=[
            pl.BlockSpec((1, _N, _D), lambda b: (b, 0, 0)),
            pl.BlockSpec((1, _N, _D), lambda b: (b, 0, 0)),
        ],
        out_shape=[
            jax.ShapeDtypeStruct((B, _N, _D), jnp.float32),
            jax.ShapeDtypeStruct((B, _N, _D), jnp.float32),
        ],
    )(X, W1, b1r, W2, b2r)

    adj = pl.pallas_call(
        _adj_kernel,
        grid=(B, _N // _RB),
        in_specs=[
            pl.BlockSpec((1, _N, _D), lambda b, i: (b, 0, 0)),
            pl.BlockSpec((1, _N, _D), lambda b, i: (b, 0, 0)),
        ],
        out_specs=pl.BlockSpec((1, _RB, _N), lambda b, i: (b, i, 0)),
        out_shape=jax.ShapeDtypeStruct((B, _N, _N), jnp.float32),
        scratch_shapes=[pltpu.VMEM((_RB, _N), jnp.float32),
                        pltpu.VMEM((_RB, 1), jnp.int32)],
    )(nv1, nv2)

    return adj


# speedup vs baseline: 33.5237x; 1.6983x over previous
import jax
import jax.numpy as jnp
from jax.experimental import pallas as pl
from jax.experimental.pallas import tpu as pltpu

_N = 2048
_F = 256
_D = 512
_K = 32
_ALPHA = 3.0
_RB = 256


def _nv_kernel(x_ref, w1_ref, b1_ref, w2_ref, b2_ref, nv1_ref, nv2_ref):
    x = x_ref[0]
    dn = (((1,), (1,)), ((), ()))
    h1 = jax.lax.dot_general(x, w1_ref[...], dn,
                             preferred_element_type=jnp.float32)
    nv1_ref[0] = jnp.tanh(_ALPHA * (h1 + b1_ref[...]))
    h2 = jax.lax.dot_general(x, w2_ref[...], dn,
                             preferred_element_type=jnp.float32)
    nv2_ref[0] = jnp.tanh(_ALPHA * (h2 + b2_ref[...]))


_C = 128
_NC = _N // _C


def _prefix_count(eq):
    eqf = jnp.where(eq, 1.0, 0.0)
    li = jax.lax.broadcasted_iota(jnp.int32, (_C, _C), 0)
    lj = jax.lax.broadcasted_iota(jnp.int32, (_C, _C), 1)
    u_incl = jnp.where(li <= lj, 1.0, 0.0)
    ci = jax.lax.broadcasted_iota(jnp.int32, (_NC, _NC), 0)
    cj = jax.lax.broadcasted_iota(jnp.int32, (_NC, _NC), 1)
    u_strict = jnp.where(ci < cj, 1.0, 0.0)
    dn = (((1,), (0,)), ((), ()))
    pcs = []
    cts = []
    for j in range(_NC):
        ej = eqf[:, j * _C:(j + 1) * _C]
        pj = jax.lax.dot_general(ej, u_incl, dn,
                                 preferred_element_type=jnp.float32)
        pcs.append(pj)
        cts.append(pj[:, _C - 1:_C])
    ct = jnp.concatenate(cts, axis=1)
    cpc = jax.lax.dot_general(ct, u_strict, dn,
                              preferred_element_type=jnp.float32)
    pc = jnp.concatenate(
        [pcs[j] + cpc[:, j:j + 1] for j in range(_NC)], axis=1)
    cnt = cpc[:, _NC - 1:_NC] + ct[:, _NC - 1:_NC]
    return pc, cnt


def _adj_kernel(nv1_ref, nv2_ref, out_ref, vbuf, rem_ref):
    i = pl.program_id(1)
    r0 = i * _RB
    nv1 = nv1_ref[0]
    nv2 = nv2_ref[0]
    nv1s = nv1_ref[0, pl.ds(r0, _RB), :]
    nv2s = nv2_ref[0, pl.ds(r0, _RB), :]
    dn = (((1,), (1,)), ((), ()))
    a = jax.lax.dot_general(nv1s, nv2, dn, preferred_element_type=jnp.float32)
    a -= jax.lax.dot_general(nv2s, nv1, dn, preferred_element_type=jnp.float32)
    adj = jnp.maximum(jnp.tanh(_ALPHA * a), 0.0)

    m = jnp.max(adj, axis=1, keepdims=True)
    eq = adj == m
    pc, cnt = _prefix_count(eq)
    take = eq & (pc <= float(_K))
    out_ref[0] = jnp.where(take, adj, 0.0)
    rem1 = _K - jnp.minimum(cnt.astype(jnp.int32), _K)
    tot1 = jnp.sum(rem1)

    @pl.when(tot1 > 0)
    def _():
        vbuf[...] = jnp.where(take, -1.0, adj)
        rem_ref[...] = rem1

    def cond(tot):
        return tot > 0

    def body(tot):
        v = vbuf[...]
        remv = rem_ref[...]
        mm = jnp.max(v, axis=1, keepdims=True)
        eq2 = v == mm
        pc2, cnt2 = _prefix_count(eq2)
        take2 = eq2 & (pc2 <= remv.astype(jnp.float32))
        out_ref[0] = jnp.where(take2, v, out_ref[0])
        rem_new = remv - jnp.minimum(cnt2.astype(jnp.int32), remv)
        tot_new = jnp.sum(rem_new)

        @pl.when(tot_new > 0)
        def _():
            vbuf[...] = jnp.where(take2, -1.0, v)
            rem_ref[...] = rem_new

        return tot_new

    jax.lax.while_loop(cond, body, tot1)


def kernel(X, W1, b1, W2, b2):
    B = X.shape[0]
    b1r = b1.reshape(1, _D)
    b2r = b2.reshape(1, _D)

    nv1, nv2 = pl.pallas_call(
        _nv_kernel,
        grid=(B,),
        in_specs=[
            pl.BlockSpec((1, _N, _F), lambda b: (b, 0, 0)),
            pl.BlockSpec((_D, _F), lambda b: (0, 0)),
            pl.BlockSpec((1, _D), lambda b: (0, 0)),
            pl.BlockSpec((_D, _F), lambda b: (0, 0)),
            pl.BlockSpec((1, _D), lambda b: (0, 0)),
        ],
        out_specs=[
            pl.BlockSpec((1, _N, _D), lambda b: (b, 0, 0)),
            pl.BlockSpec((1, _N, _D), lambda b: (b, 0, 0)),
        ],
        out_shape=[
            jax.ShapeDtypeStruct((B, _N, _D), jnp.float32),
            jax.ShapeDtypeStruct((B, _N, _D), jnp.float32),
        ],
    )(X, W1, b1r, W2, b2r)

    adj = pl.pallas_call(
        _adj_kernel,
        grid=(B, _N // _RB),
        in_specs=[
            pl.BlockSpec((1, _N, _D), lambda b, i: (b, 0, 0)),
            pl.BlockSpec((1, _N, _D), lambda b, i: (b, 0, 0)),
        ],
        out_specs=pl.BlockSpec((1, _RB, _N), lambda b, i: (b, i, 0)),
        out_shape=jax.ShapeDtypeStruct((B, _N, _N), jnp.float32),
        scratch_shapes=[pltpu.VMEM((_RB, _N), jnp.float32),
                        pltpu.VMEM((_RB, 1), jnp.int32)],
    )(nv1, nv2)

    return adj


# speedup vs baseline: 43.0845x; 1.2852x over previous
import jax
import jax.numpy as jnp
from jax.experimental import pallas as pl
from jax.experimental.pallas import tpu as pltpu

_N = 2048
_F = 256
_D = 512
_K = 32
_ALPHA = 3.0
_RB = 512
_C = 128
_NC = _N // _C


def _select_pass(v, rem_f):
    m = jnp.max(v, axis=1, keepdims=True)
    li = jax.lax.broadcasted_iota(jnp.int32, (_C, _C), 0)
    lj = jax.lax.broadcasted_iota(jnp.int32, (_C, _C), 1)
    u_incl = jnp.where(li <= lj, 1.0, 0.0)
    ci = jax.lax.broadcasted_iota(jnp.int32, (_NC, _NC), 0)
    cj = jax.lax.broadcasted_iota(jnp.int32, (_NC, _NC), 1)
    u_strict = jnp.where(ci < cj, 1.0, 0.0)
    dn = (((1,), (0,)), ((), ()))
    eqs = []
    pjs = []
    for j in range(_NC):
        ej = v[:, j * _C:(j + 1) * _C] == m
        pj = jax.lax.dot_general(jnp.where(ej, 1.0, 0.0), u_incl, dn,
                                 preferred_element_type=jnp.float32)
        eqs.append(ej)
        pjs.append(pj)
    ct = jnp.concatenate([pj[:, _C - 1:_C] for pj in pjs], axis=1)
    cpc = jax.lax.dot_general(ct, u_strict, dn,
                              preferred_element_type=jnp.float32)
    cnt = cpc[:, _NC - 1:_NC] + ct[:, _NC - 1:_NC]
    takes = []
    for j in range(_NC):
        takes.append(eqs[j] & (pjs[j] <= rem_f - cpc[:, j:j + 1]))
    return takes, cnt


def _graph_kernel(x_ref, w1_ref, b1_ref, w2_ref, b2_ref, out_ref,
                  nv1_s, nv2_s, vbuf, rem_ref):
    i = pl.program_id(1)

    @pl.when(i == 0)
    def _():
        x = x_ref[0]
        dnf = (((1,), (1,)), ((), ()))
        h1 = jax.lax.dot_general(x, w1_ref[...], dnf,
                                 preferred_element_type=jnp.float32)
        nv1_s[...] = jnp.tanh(_ALPHA * (h1 + b1_ref[...]))
        h2 = jax.lax.dot_general(x, w2_ref[...], dnf,
                                 preferred_element_type=jnp.float32)
        nv2_s[...] = jnp.tanh(_ALPHA * (h2 + b2_ref[...]))

    r0 = i * _RB
    nv1 = nv1_s[...]
    nv2 = nv2_s[...]
    nv1r = nv1_s[pl.ds(r0, _RB), :]
    nv2r = nv2_s[pl.ds(r0, _RB), :]
    dnd = (((1,), (1,)), ((), ()))
    a = jax.lax.dot_general(nv1r, nv2, dnd, preferred_element_type=jnp.float32)
    a -= jax.lax.dot_general(nv2r, nv1, dnd, preferred_element_type=jnp.float32)
    adj = jnp.maximum(jnp.tanh(_ALPHA * a), 0.0)

    takes, cnt = _select_pass(adj, jnp.full((_RB, 1), float(_K), jnp.float32))
    for j in range(_NC):
        sl = slice(j * _C, (j + 1) * _C)
        out_ref[0, :, sl] = jnp.where(takes[j], adj[:, sl], 0.0)
    rem1 = _K - jnp.minimum(cnt.astype(jnp.int32), _K)
    tot1 = jnp.sum(rem1)

    @pl.when(tot1 > 0)
    def _():
        for j in range(_NC):
            sl = slice(j * _C, (j + 1) * _C)
            vbuf[:, sl] = jnp.where(takes[j], -1.0, adj[:, sl])
        rem_ref[...] = rem1

    def cond(tot):
        return tot > 0

    def body(tot):
        v = vbuf[...]
        remv = rem_ref[...]
        takes2, cnt2 = _select_pass(v, remv.astype(jnp.float32))
        for j in range(_NC):
            sl = slice(j * _C, (j + 1) * _C)
            out_ref[0, :, sl] = jnp.where(takes2[j], v[:, sl],
                                          out_ref[0, :, sl])
        rem_new = remv - jnp.minimum(cnt2.astype(jnp.int32), remv)
        tot_new = jnp.sum(rem_new)

        @pl.when(tot_new > 0)
        def _():
            for j in range(_NC):
                sl = slice(j * _C, (j + 1) * _C)
                vbuf[:, sl] = jnp.where(takes2[j], -1.0, v[:, sl])
            rem_ref[...] = rem_new

        return tot_new

    jax.lax.while_loop(cond, body, tot1)


def kernel(X, W1, b1, W2, b2):
    B = X.shape[0]
    b1r = b1.reshape(1, _D)
    b2r = b2.reshape(1, _D)

    adj = pl.pallas_call(
        _graph_kernel,
        grid=(B, _N // _RB),
        in_specs=[
            pl.BlockSpec((1, _N, _F), lambda b, i: (b, 0, 0)),
            pl.BlockSpec((_D, _F), lambda b, i: (0, 0)),
            pl.BlockSpec((1, _D), lambda b, i: (0, 0)),
            pl.BlockSpec((_D, _F), lambda b, i: (0, 0)),
            pl.BlockSpec((1, _D), lambda b, i: (0, 0)),
        ],
        out_specs=pl.BlockSpec((1, _RB, _N), lambda b, i: (b, i, 0)),
        out_shape=jax.ShapeDtypeStruct((B, _N, _N), jnp.float32),
        scratch_shapes=[
            pltpu.VMEM((_N, _D), jnp.float32),
            pltpu.VMEM((_N, _D), jnp.float32),
            pltpu.VMEM((_RB, _N), jnp.float32),
            pltpu.VMEM((_RB, 1), jnp.int32),
        ],
    )(X, W1, b1r, W2, b2r)

    return adj


# speedup vs baseline: 43.5389x; 1.0105x over previous
import jax
import jax.numpy as jnp
from jax.experimental import pallas as pl
from jax.experimental.pallas import tpu as pltpu

_N = 2048
_F = 256
_D = 512
_K = 32
_ALPHA = 3.0
_RB = 512
_C = 128
_NC = _N // _C


def _select_pass(v, rem_f):
    m = jnp.max(v, axis=1, keepdims=True)
    li = jax.lax.broadcasted_iota(jnp.int32, (_C, _C), 0)
    lj = jax.lax.broadcasted_iota(jnp.int32, (_C, _C), 1)
    u_incl = jnp.where(li <= lj, 1.0, 0.0).astype(jnp.bfloat16)
    ci = jax.lax.broadcasted_iota(jnp.int32, (_NC, _NC), 0)
    cj = jax.lax.broadcasted_iota(jnp.int32, (_NC, _NC), 1)
    u_strict = jnp.where(ci < cj, 1.0, 0.0).astype(jnp.bfloat16)
    dn = (((1,), (0,)), ((), ()))
    pjs = []
    for j in range(_NC):
        ej = v[:, j * _C:(j + 1) * _C] == m
        eqf = jnp.where(ej, 1.0, 0.0).astype(jnp.bfloat16)
        pjs.append(jax.lax.dot_general(eqf, u_incl, dn,
                                       preferred_element_type=jnp.float32))
    ct = jnp.concatenate([pj[:, _C - 1:_C] for pj in pjs], axis=1)
    cpc = jax.lax.dot_general(ct.astype(jnp.bfloat16), u_strict, dn,
                              preferred_element_type=jnp.float32)
    cnt = cpc[:, _NC - 1:_NC] + ct[:, _NC - 1:_NC]
    thrs = [rem_f - cpc[:, j:j + 1] for j in range(_NC)]
    return m, thrs, pjs, cnt


def _graph_kernel(x_ref, w1_ref, b1_ref, w2_ref, b2_ref, out_ref,
                  nv1_s, nv2_s, vbuf, rem_ref):
    i = pl.program_id(1)

    @pl.when(i == 0)
    def _():
        x = x_ref[0]
        dnf = (((1,), (1,)), ((), ()))
        h1 = jax.lax.dot_general(x, w1_ref[...], dnf,
                                 preferred_element_type=jnp.float32)
        nv1_s[...] = jnp.tanh(_ALPHA * (h1 + b1_ref[...]))
        h2 = jax.lax.dot_general(x, w2_ref[...], dnf,
                                 preferred_element_type=jnp.float32)
        nv2_s[...] = jnp.tanh(_ALPHA * (h2 + b2_ref[...]))

    r0 = i * _RB
    nv1 = nv1_s[...]
    nv2 = nv2_s[...]
    nv1r = nv1_s[pl.ds(r0, _RB), :]
    nv2r = nv2_s[pl.ds(r0, _RB), :]
    dnd = (((1,), (1,)), ((), ()))
    a = jax.lax.dot_general(nv1r, nv2, dnd, preferred_element_type=jnp.float32)
    a -= jax.lax.dot_general(nv2r, nv1, dnd, preferred_element_type=jnp.float32)
    adj = jnp.maximum(jnp.tanh(_ALPHA * a), 0.0)

    m, thrs, pjs, cnt = _select_pass(
        adj, jnp.full((_RB, 1), float(_K), jnp.float32))
    for j in range(_NC):
        sl = slice(j * _C, (j + 1) * _C)
        aj = adj[:, sl]
        out_ref[0, :, sl] = jnp.where((aj == m) & (pjs[j] <= thrs[j]),
                                      aj, 0.0)
    rem1 = _K - jnp.minimum(cnt.astype(jnp.int32), _K)
    tot1 = jnp.sum(rem1)

    @pl.when(tot1 > 0)
    def _():
        for j in range(_NC):
            sl = slice(j * _C, (j + 1) * _C)
            aj = adj[:, sl]
            vbuf[:, sl] = jnp.where((aj == m) & (pjs[j] <= thrs[j]),
                                    -1.0, aj)
        rem_ref[...] = rem1

    def cond(tot):
        return tot > 0

    def body(tot):
        v = vbuf[...]
        remv = rem_ref[...]
        m2, thrs2, pjs2, cnt2 = _select_pass(v, remv.astype(jnp.float32))
        for j in range(_NC):
            sl = slice(j * _C, (j + 1) * _C)
            vj = v[:, sl]
            take2 = (vj == m2) & (pjs2[j] <= thrs2[j])
            out_ref[0, :, sl] = jnp.where(take2, vj, out_ref[0, :, sl])
        rem_new = remv - jnp.minimum(cnt2.astype(jnp.int32), remv)
        tot_new = jnp.sum(rem_new)

        @pl.when(tot_new > 0)
        def _():
            for j in range(_NC):
                sl = slice(j * _C, (j + 1) * _C)
                vj = v[:, sl]
                vbuf[:, sl] = jnp.where((vj == m2) & (pjs2[j] <= thrs2[j]),
                                        -1.0, vj)
            rem_ref[...] = rem_new

        return tot_new

    jax.lax.while_loop(cond, body, tot1)


def kernel(X, W1, b1, W2, b2):
    B = X.shape[0]
    b1r = b1.reshape(1, _D)
    b2r = b2.reshape(1, _D)

    adj = pl.pallas_call(
        _graph_kernel,
        grid=(B, _N // _RB),
        in_specs=[
            pl.BlockSpec((1, _N, _F), lambda b, i: (b, 0, 0)),
            pl.BlockSpec((_D, _F), lambda b, i: (0, 0)),
            pl.BlockSpec((1, _D), lambda b, i: (0, 0)),
            pl.BlockSpec((_D, _F), lambda b, i: (0, 0)),
            pl.BlockSpec((1, _D), lambda b, i: (0, 0)),
        ],
        out_specs=pl.BlockSpec((1, _RB, _N), lambda b, i: (b, i, 0)),
        out_shape=jax.ShapeDtypeStruct((B, _N, _N), jnp.float32),
        scratch_shapes=[
            pltpu.VMEM((_N, _D), jnp.float32),
            pltpu.VMEM((_N, _D), jnp.float32),
            pltpu.VMEM((_RB, _N), jnp.float32),
            pltpu.VMEM((_RB, 1), jnp.int32),
        ],
    )(X, W1, b1r, W2, b2r)

    return adj
